# baseline (speedup 1.0000x reference)
import functools

import jax
import jax.numpy as jnp
from jax import lax
from jax.experimental import pallas as pl
from jax.experimental.pallas import tpu as pltpu
from jax.experimental.pallas import tpu_sc as plsc

N = 10000
E = 320000
NFEAT = 128
NHID = 128
NCLASS = 64

NC = 2
NS = 16
NW = NC * NS
C1 = 64
C2 = 128
CD = 128
E_PAD = 331776
EW_WORK = E_PAD // NW
NCHD = E_PAD // CD // NS
NROWS_PAD = 10240
ZRO = NROWS_PAD // NS

MROWS = 1000


def _when(cond, fn):
    if isinstance(cond, bool):
        if cond:
            fn()
    else:
        pl.when(cond)(fn)


def _rsqrt16(d):
    i = plsc.bitcast(d, jnp.int32)
    i = jnp.int32(0x5F3759DF) - (i >> 1)
    y = plsc.bitcast(i, jnp.float32)
    for _ in range(4):
        y = y * (1.5 - 0.5 * d * y * y)
    return y


def _scale_rows(rows_v, normv, cc, d):
    @pl.loop(0, cc // 4)
    def _e(eb):
        for u in range(4):
            e = eb * 4 + u
            spl = plsc.load_gather(normv, [jnp.full((16,), e, jnp.int32)])
            for k in range(d // 16):
                sl = pl.ds(k * 16, 16)
                rows_v[e, sl] = rows_v[e, sl] * spl


_Z16 = functools.partial(jnp.zeros, (16,), jnp.float32)


def _idx_start(row_ref, col_ref, aux_ref, rowv, colv, auxv, sem, base, cc):
    pltpu.async_copy(row_ref.at[pl.ds(base, cc)], rowv, sem)
    pltpu.async_copy(col_ref.at[pl.ds(base, cc)], colv, sem)
    pltpu.async_copy(aux_ref.at[pl.ds(base, cc)], auxv, sem)


def _idx_wait(row_ref, col_ref, aux_ref, rowv, colv, auxv, sem, base, cc):
    pltpu.make_async_copy(row_ref.at[pl.ds(base, cc)], rowv, sem).wait()
    pltpu.make_async_copy(col_ref.at[pl.ds(base, cc)], colv, sem).wait()
    pltpu.make_async_copy(aux_ref.at[pl.ds(base, cc)], auxv, sem).wait()


def _make_agg_loop(row_ref, col_ref, aux_ref, h_ref, acc_s, rowv, colv,
                   auxv, rows, si, sg, ss, w, compute, cc):
    nch = EW_WORK // cc
    main = (nch - 1) // 4 * 4
    tail = nch - main
    w_base = w * EW_WORK

    def idx_start(slot, j):
        _idx_start(row_ref, col_ref, aux_ref, rowv[slot], colv[slot],
                   auxv[slot], si[slot], w_base + j * cc, cc)

    def idx_wait(slot, j):
        _idx_wait(row_ref, col_ref, aux_ref, rowv[slot], colv[slot],
                  auxv[slot], si[slot], w_base + j * cc, cc)

    def gather_start(slot):
        pltpu.async_copy(h_ref.at[rowv[slot]], rows[slot], sg[slot])

    def gather_wait(slot):
        pltpu.make_async_copy(h_ref.at[rowv[slot]], rows[slot],
                              sg[slot]).wait()

    def scat_start(slot):
        pltpu.async_copy(rows[slot], acc_s.at[colv[slot]], ss[slot],
                         add=True)

    def scat_wait(slot):
        pltpu.make_async_copy(rows[slot], acc_s.at[colv[slot]],
                              ss[slot]).wait()

    idx_start(0, 0)
    idx_start(1, 1)
    idx_wait(0, 0)
    gather_start(0)

    @pl.loop(0, main // 4)
    def _blk(blk):
        for b in range(4):
            jj = blk * 4 + b
            nslot = (b + 1) % 4
            idx_wait(nslot, jj + 1)
            _when(jj >= 2, lambda: scat_wait((b + 2) % 4))
            gather_start(nslot)
            _when(jj + 2 < nch, lambda: idx_start((b + 2) % 4, jj + 2))
            gather_wait(b)
            compute(b, jj, w_base + jj * cc)
            scat_start(b)

    scat_wait(2)
    scat_wait(3)
    gather_wait(0)
    if tail == 2:
        idx_wait(1, main + 1)
        gather_start(1)
    compute(0, main, w_base + main * cc)
    if tail == 2:
        pltpu.async_copy(rows[0], acc_s.at[colv[0]], ss[0], add=True)
        gather_wait(1)
        compute(1, main + 1, w_base + (main + 1) * cc)
        pltpu.sync_copy(rows[1], acc_s.at[colv[1]], add=True)
        scat_wait(0)
    else:
        pltpu.sync_copy(rows[0], acc_s.at[colv[0]], add=True)


NCHD0 = E_PAD // CD // NW


def _deg_body(col_ref, ew_ref, degp_ref,
              deg_s, colD0, colD1, colD2, ewD0, ewD1, ewD2, zdeg,
              si0, si1, si2, ss0, ss1, ss2):
    c = lax.axis_index("c")
    s = lax.axis_index("s")
    w = s * NC + c
    colD = (colD0, colD1, colD2)
    ewD = (ewD0, ewD1, ewD2)
    si = (si0, si1, si2)
    ss = (ss0, ss1, ss2)

    for k in range(ZRO // 16):
        zdeg[pl.ds(k * 16, 16)] = _Z16()
    pltpu.sync_copy(zdeg, deg_s.at[pl.ds(s * ZRO, ZRO)])
    plsc.subcore_barrier()

    d_base = w * NCHD0 * CD

    def didx_start(slot, j):
        pltpu.async_copy(col_ref.at[pl.ds(d_base + j * CD, CD)],
                         colD[slot], si[slot])
        pltpu.async_copy(ew_ref.at[pl.ds(d_base + j * CD, CD)],
                         ewD[slot], si[slot])

    def didx_wait(slot, j):
        pltpu.make_async_copy(col_ref.at[pl.ds(d_base + j * CD, CD)],
                              colD[slot], si[slot]).wait()
        pltpu.make_async_copy(ew_ref.at[pl.ds(d_base + j * CD, CD)],
                              ewD[slot], si[slot]).wait()

    def dscat_wait(slot):
        pltpu.make_async_copy(ewD[slot], deg_s.at[colD[slot]],
                              ss[slot]).wait()

    didx_start(0, 0)

    @pl.loop(0, NCHD0 // 3)
    def _dblk(blk):
        for m in range(3):
            jj = blk * 3 + m
            nslot = (m + 1) % 3

            def _advance():
                _when(jj >= 2, lambda: dscat_wait(nslot))
                didx_start(nslot, jj + 1)

            _when(jj + 1 < NCHD0, _advance)
            didx_wait(m, jj)
            pltpu.async_copy(ewD[m], deg_s.at[colD[m]], ss[m], add=True)

    dscat_wait(0)
    dscat_wait(1)
    dscat_wait(2)
    plsc.subcore_barrier()
    pltpu.sync_copy(deg_s.at[pl.ds(s * ZRO, ZRO)],
                    degp_ref.at[pl.ds(c * NROWS_PAD + s * ZRO, ZRO)])


def _layer1_body(row_ref, col_ref, ew_ref, h_ref, degp_ref, agg_ref,
                 acc_s, dinvv,
                 rowv0, rowv1, rowv2, rowv3,
                 colv0, colv1, colv2, colv3,
                 auxv0, auxv1, auxv2, auxv3,
                 normv0, normv1,
                 rows0, rows1, rows2, rows3,
                 zrows, zdeg, zdeg2,
                 si0, si1, si2, si3, sg0, sg1, sg2, sg3,
                 ss0, ss1, ss2, ss3):
    c = lax.axis_index("c")
    s = lax.axis_index("s")
    w = s * NC + c
    rowv = (rowv0, rowv1, rowv2, rowv3)
    colv = (colv0, colv1, colv2, colv3)
    auxv = (auxv0, auxv1, auxv2, auxv3)
    normv = (normv0, normv1)
    rows = (rows0, rows1, rows2, rows3)
    si = (si0, si1, si2, si3)
    sg = (sg0, sg1, sg2, sg3)
    ss = (ss0, ss1, ss2, ss3)

    for r in range(16):
        for k in range(NHID // 16):
            zrows[r, pl.ds(k * 16, 16)] = _Z16()

    for k in range(ZRO // 16):
        pltpu.async_copy(zrows, acc_s.at[pl.ds(s * ZRO + k * 16, 16)], sg0)

    @pl.loop(0, NS)
    def _piece(p):
        off = p * ZRO
        pltpu.sync_copy(degp_ref.at[pl.ds(off, ZRO)], zdeg)
        pltpu.sync_copy(degp_ref.at[pl.ds(NROWS_PAD + off, ZRO)], zdeg2)

        @pl.loop(0, ZRO // 16)
        def _newton(i):
            sl = pl.ds(i * 16, 16)
            dinvv[pl.ds(off + i * 16, 16)] = _rsqrt16(zdeg[sl] + zdeg2[sl])

    for k in range(ZRO // 16):
        pltpu.make_async_copy(zrows, acc_s.at[pl.ds(s * ZRO + k * 16, 16)],
                              sg0).wait()
    plsc.subcore_barrier()

    def compute(slot, jj, base):
        rv = rows[slot]
        nv = normv[slot % 2]
        for gi in range(C1 // 16):
            sl = pl.ds(gi * 16, 16)
            dr = plsc.load_gather(dinvv, [rowv[slot][sl]])
            dc = plsc.load_gather(dinvv, [colv[slot][sl]])
            nv[sl] = dr * auxv[slot][sl] * dc
        _scale_rows(rv, nv, C1, NHID)

    _make_agg_loop(row_ref, col_ref, ew_ref, h_ref, acc_s, rowv, colv,
                   auxv, rows, si, sg, ss, w, compute, C1)

    plsc.subcore_barrier()
    pltpu.sync_copy(acc_s.at[pl.ds(s * ZRO, ZRO)],
                    agg_ref.at[c, pl.ds(s * ZRO, ZRO)])


def _layer2_body(row_ref, col_ref, ew_ref, h_ref, degp_ref, agg_ref,
                 acc_s, dinvv,
                 rowv0, rowv1, rowv2, rowv3,
                 colv0, colv1, colv2, colv3,
                 auxv0, auxv1, auxv2, auxv3,
                 rows0, rows1, rows2, rows3,
                 zrows, zdeg, zdeg2,
                 si0, si1, si2, si3, sg0, sg1, sg2, sg3,
                 ss0, ss1, ss2, ss3):
    c = lax.axis_index("c")
    s = lax.axis_index("s")
    w = s * NC + c
    rowv = (rowv0, rowv1, rowv2, rowv3)
    colv = (colv0, colv1, colv2, colv3)
    auxv = (auxv0, auxv1, auxv2, auxv3)
    rows = (rows0, rows1, rows2, rows3)
    si = (si0, si1, si2, si3)
    sg = (sg0, sg1, sg2, sg3)
    ss = (ss0, ss1, ss2, ss3)

    for r in range(16):
        for k in range(NCLASS // 16):
            zrows[r, pl.ds(k * 16, 16)] = _Z16()
    for k in range(ZRO // 16):
        pltpu.async_copy(zrows, acc_s.at[pl.ds(s * ZRO + k * 16, 16)], sg0)

    @pl.loop(0, NS)
    def _piece(p):
        off = p * ZRO
        pltpu.sync_copy(degp_ref.at[pl.ds(off, ZRO)], zdeg)
        pltpu.sync_copy(degp_ref.at[pl.ds(NROWS_PAD + off, ZRO)], zdeg2)

        @pl.loop(0, ZRO // 16)
        def _newton(i):
            sl = pl.ds(i * 16, 16)
            dinvv[pl.ds(off + i * 16, 16)] = _rsqrt16(zdeg[sl] + zdeg2[sl])

    for k in range(ZRO // 16):
        pltpu.make_async_copy(zrows, acc_s.at[pl.ds(s * ZRO + k * 16, 16)],
                              sg0).wait()
    plsc.subcore_barrier()

    def compute(slot, jj, base):
        rv = rows[slot]
        nv = auxv[slot]
        for gi in range(C2 // 16):
            sl = pl.ds(gi * 16, 16)
            dr = plsc.load_gather(dinvv, [rowv[slot][sl]])
            dc = plsc.load_gather(dinvv, [colv[slot][sl]])
            nv[sl] = dr * nv[sl] * dc
        _scale_rows(rv, nv, C2, NCLASS)

    _make_agg_loop(row_ref, col_ref, ew_ref, h_ref, acc_s, rowv, colv,
                   auxv, rows, si, sg, ss, w, compute, C2)

    plsc.subcore_barrier()
    pltpu.sync_copy(acc_s.at[pl.ds(s * ZRO, ZRO)],
                    agg_ref.at[c, pl.ds(s * ZRO, ZRO)])


_sc_mesh = plsc.VectorSubcoreMesh(core_axis_name="c", subcore_axis_name="s")

_DMA = pltpu.SemaphoreType.DMA

_sc_deg = functools.partial(
    pl.kernel,
    out_type=jax.ShapeDtypeStruct((NC * NROWS_PAD,), jnp.float32),
    mesh=_sc_mesh,
    compiler_params=pltpu.CompilerParams(needs_layout_passes=False),
    scratch_types=[
        pltpu.VMEM_SHARED((NROWS_PAD,), jnp.float32),
    ] + [pltpu.VMEM((CD,), jnp.int32)] * 3
      + [pltpu.VMEM((CD,), jnp.float32)] * 3
      + [pltpu.VMEM((ZRO,), jnp.float32)]
      + [_DMA] * 6,
)(_deg_body)

_sc_layer1 = functools.partial(
    pl.kernel,
    out_type=jax.ShapeDtypeStruct((NC, NROWS_PAD, NHID), jnp.float32),
    mesh=_sc_mesh,
    compiler_params=pltpu.CompilerParams(needs_layout_passes=False),
    scratch_types=[
        pltpu.VMEM_SHARED((NROWS_PAD, NHID), jnp.float32),
        pltpu.VMEM((NROWS_PAD,), jnp.float32),
    ] + [pltpu.VMEM((C1,), jnp.int32)] * 4
      + [pltpu.VMEM((C1,), jnp.int32)] * 4
      + [pltpu.VMEM((C1,), jnp.float32)] * 4
      + [pltpu.VMEM((C1,), jnp.float32)] * 2
      + [pltpu.VMEM((C1, NHID), jnp.float32)] * 4
      + [pltpu.VMEM((16, NHID), jnp.float32),
         pltpu.VMEM((ZRO,), jnp.float32),
         pltpu.VMEM((ZRO,), jnp.float32)]
      + [_DMA] * 12,
)(_layer1_body)

_sc_layer2 = functools.partial(
    pl.kernel,
    out_type=jax.ShapeDtypeStruct((NC, NROWS_PAD, NCLASS), jnp.float32),
    mesh=_sc_mesh,
    compiler_params=pltpu.CompilerParams(needs_layout_passes=False,
                                         use_tc_tiling_on_sc=False),
    scratch_types=[
        pltpu.VMEM_SHARED((NROWS_PAD, NCLASS), jnp.float32),
        pltpu.VMEM((NROWS_PAD,), jnp.float32),
    ] + [pltpu.VMEM((C2,), jnp.int32)] * 4
      + [pltpu.VMEM((C2,), jnp.int32)] * 4
      + [pltpu.VMEM((C2,), jnp.float32)] * 4
      + [pltpu.VMEM((C2, NCLASS), jnp.float32)] * 4
      + [pltpu.VMEM((16, NCLASS), jnp.float32),
         pltpu.VMEM((ZRO,), jnp.float32),
         pltpu.VMEM((ZRO,), jnp.float32)]
      + [_DMA] * 12,
)(_layer2_body)


def _mm_body(x_ref, w_ref, o_ref):
    o_ref[...] = jnp.dot(x_ref[...], w_ref[...],
                         preferred_element_type=jnp.float32)


def _matmul(x, w):
    m, k = x.shape
    n = w.shape[1]
    return pl.pallas_call(
        _mm_body,
        grid=(m // MROWS,),
        in_specs=[pl.BlockSpec((MROWS, k), lambda i: (i, 0)),
                  pl.BlockSpec((k, n), lambda i: (0, 0))],
        out_specs=pl.BlockSpec((MROWS, n), lambda i: (i, 0)),
        out_shape=jax.ShapeDtypeStruct((m, n), jnp.float32),
    )(x, w)


def _relu_mm_body(p0_ref, p1_ref, b_ref, w_ref, o_ref):
    h = jnp.maximum(p0_ref[...] + p1_ref[...] + b_ref[...], 0.0)
    o_ref[...] = jnp.dot(h, w_ref[...], preferred_element_type=jnp.float32)


def _relu_mm(p0, p1, b, w):
    m, k = p0.shape
    n = w.shape[1]
    return pl.pallas_call(
        _relu_mm_body,
        grid=(m // MROWS,),
        in_specs=[pl.BlockSpec((MROWS, k), lambda i: (i, 0)),
                  pl.BlockSpec((MROWS, k), lambda i: (i, 0)),
                  pl.BlockSpec((1, k), lambda i: (0, 0)),
                  pl.BlockSpec((k, n), lambda i: (0, 0))],
        out_specs=pl.BlockSpec((MROWS, n), lambda i: (i, 0)),
        out_shape=jax.ShapeDtypeStruct((m, n), jnp.float32),
    )(p0, p1, b, w)


def _lsm_body(p0_ref, p1_ref, b_ref, o_ref):
    sv = p0_ref[...] + p1_ref[...] + b_ref[...]
    m = jnp.max(sv, axis=1, keepdims=True)
    t = sv - m
    o_ref[...] = t - jnp.log(jnp.sum(jnp.exp(t), axis=1, keepdims=True))


def _logsoftmax(p0, p1, b):
    m, n = p0.shape
    return pl.pallas_call(
        _lsm_body,
        grid=(m // MROWS,),
        in_specs=[pl.BlockSpec((MROWS, n), lambda i: (i, 0)),
                  pl.BlockSpec((MROWS, n), lambda i: (i, 0)),
                  pl.BlockSpec((1, n), lambda i: (0, 0))],
        out_specs=pl.BlockSpec((MROWS, n), lambda i: (i, 0)),
        out_shape=jax.ShapeDtypeStruct((m, n), jnp.float32),
    )(p0, p1, b)


def kernel(x, edge_index, edge_weight, W1, b1, W2, b2):
    row = edge_index[0].astype(jnp.int32)
    col = edge_index[1].astype(jnp.int32)
    loop_idx = jnp.arange(N, dtype=jnp.int32)
    pad = E_PAD - E - N
    zpad = jnp.zeros((pad,), jnp.int32)
    row_ext = jnp.concatenate([row, loop_idx, zpad])
    col_ext = jnp.concatenate([col, loop_idx, zpad])
    ew_ext = jnp.concatenate([edge_weight.astype(jnp.float32),
                              jnp.ones((N,), jnp.float32),
                              jnp.zeros((pad,), jnp.float32)])

    degp = _sc_deg(col_ext, ew_ext)
    h1 = _matmul(x, W1)
    agg1 = _sc_layer1(row_ext, col_ext, ew_ext, h1, degp)
    h2 = _relu_mm(agg1[0, :N], agg1[1, :N], b1.reshape(1, NHID), W2)
    agg2 = _sc_layer2(row_ext, col_ext, ew_ext, h2, degp)
    return _logsoftmax(agg2[0, :N], agg2[1, :N], b2.reshape(1, NCLASS))


# speedup vs baseline: 1.1196x; 1.1196x over previous
import functools

import jax
import jax.numpy as jnp
from jax import lax
from jax.experimental import pallas as pl
from jax.experimental.pallas import tpu as pltpu
from jax.experimental.pallas import tpu_sc as plsc

N = 10000
E = 320000
NFEAT = 128
NHID = 128
NCLASS = 64

NC = 2
NS = 16
NW = NC * NS
C1 = 64
C2 = 128
CD = 128
E_PAD = 331776
EW_WORK = E_PAD // NW
NCHD = E_PAD // CD // NS
NROWS_PAD = 10240
ZRO = NROWS_PAD // NS

MROWS = 1000


def _when(cond, fn):
    if isinstance(cond, bool):
        if cond:
            fn()
    else:
        pl.when(cond)(fn)


def _rsqrt16(d):
    i = plsc.bitcast(d, jnp.int32)
    i = jnp.int32(0x5F3759DF) - (i >> 1)
    y = plsc.bitcast(i, jnp.float32)
    for _ in range(4):
        y = y * (1.5 - 0.5 * d * y * y)
    return y


def _scale_rows(rows_v, normv, cc, d):
    @pl.loop(0, cc // 4)
    def _e(eb):
        for u in range(4):
            e = eb * 4 + u
            spl = plsc.load_gather(normv, [jnp.full((16,), e, jnp.int32)])
            for k in range(d // 16):
                sl = pl.ds(k * 16, 16)
                rows_v[e, sl] = rows_v[e, sl] * spl


_Z16 = functools.partial(jnp.zeros, (16,), jnp.float32)


def _idx_start(row_ref, col_ref, aux_ref, rowv, colv, auxv, sem, base, cc):
    pltpu.async_copy(row_ref.at[pl.ds(base, cc)], rowv, sem)
    pltpu.async_copy(col_ref.at[pl.ds(base, cc)], colv, sem)
    pltpu.async_copy(aux_ref.at[pl.ds(base, cc)], auxv, sem)


def _idx_wait(row_ref, col_ref, aux_ref, rowv, colv, auxv, sem, base, cc):
    pltpu.make_async_copy(row_ref.at[pl.ds(base, cc)], rowv, sem).wait()
    pltpu.make_async_copy(col_ref.at[pl.ds(base, cc)], colv, sem).wait()
    pltpu.make_async_copy(aux_ref.at[pl.ds(base, cc)], auxv, sem).wait()


def _make_agg_loop(row_ref, col_ref, aux_ref, h_ref, acc_s, rowv, colv,
                   auxv, rows, si, sg, ss, w, compute, cc):
    nch = EW_WORK // cc
    main = (nch - 1) // 4 * 4
    tail = nch - main
    w_base = w * EW_WORK

    def idx_start(slot, j):
        _idx_start(row_ref, col_ref, aux_ref, rowv[slot], colv[slot],
                   auxv[slot], si[slot], w_base + j * cc, cc)

    def idx_wait(slot, j):
        _idx_wait(row_ref, col_ref, aux_ref, rowv[slot], colv[slot],
                  auxv[slot], si[slot], w_base + j * cc, cc)

    def gather_start(slot):
        pltpu.async_copy(h_ref.at[rowv[slot]], rows[slot], sg[slot])

    def gather_wait(slot):
        pltpu.make_async_copy(h_ref.at[rowv[slot]], rows[slot],
                              sg[slot]).wait()

    def scat_start(slot):
        pltpu.async_copy(rows[slot], acc_s.at[colv[slot]], ss[slot],
                         add=True)

    def scat_wait(slot):
        pltpu.make_async_copy(rows[slot], acc_s.at[colv[slot]],
                              ss[slot]).wait()

    idx_start(0, 0)
    idx_start(1, 1)
    idx_wait(0, 0)
    gather_start(0)

    @pl.loop(0, main // 4)
    def _blk(blk):
        for b in range(4):
            jj = blk * 4 + b
            nslot = (b + 1) % 4
            idx_wait(nslot, jj + 1)
            _when(jj >= 2, lambda: scat_wait((b + 2) % 4))
            gather_start(nslot)
            _when(jj + 2 < nch, lambda: idx_start((b + 2) % 4, jj + 2))
            gather_wait(b)
            compute(b, jj, w_base + jj * cc)
            scat_start(b)

    scat_wait(2)
    scat_wait(3)
    gather_wait(0)
    if tail == 2:
        idx_wait(1, main + 1)
        gather_start(1)
    compute(0, main, w_base + main * cc)
    if tail == 2:
        pltpu.async_copy(rows[0], acc_s.at[colv[0]], ss[0], add=True)
        gather_wait(1)
        compute(1, main + 1, w_base + (main + 1) * cc)
        pltpu.sync_copy(rows[1], acc_s.at[colv[1]], add=True)
        scat_wait(0)
    else:
        pltpu.sync_copy(rows[0], acc_s.at[colv[0]], add=True)


NCHD0 = E_PAD // CD // NW


def _deg_body(col_ref, ew_ref, degp_ref,
              deg_s, colD0, colD1, colD2, ewD0, ewD1, ewD2, zdeg,
              si0, si1, si2, ss0, ss1, ss2):
    c = lax.axis_index("c")
    s = lax.axis_index("s")
    w = s * NC + c
    colD = (colD0, colD1, colD2)
    ewD = (ewD0, ewD1, ewD2)
    si = (si0, si1, si2)
    ss = (ss0, ss1, ss2)

    for k in range(ZRO // 16):
        zdeg[pl.ds(k * 16, 16)] = _Z16()
    pltpu.sync_copy(zdeg, deg_s.at[pl.ds(s * ZRO, ZRO)])
    plsc.subcore_barrier()

    d_base = w * NCHD0 * CD

    def didx_start(slot, j):
        pltpu.async_copy(col_ref.at[pl.ds(d_base + j * CD, CD)],
                         colD[slot], si[slot])
        pltpu.async_copy(ew_ref.at[pl.ds(d_base + j * CD, CD)],
                         ewD[slot], si[slot])

    def didx_wait(slot, j):
        pltpu.make_async_copy(col_ref.at[pl.ds(d_base + j * CD, CD)],
                              colD[slot], si[slot]).wait()
        pltpu.make_async_copy(ew_ref.at[pl.ds(d_base + j * CD, CD)],
                              ewD[slot], si[slot]).wait()

    def dscat_wait(slot):
        pltpu.make_async_copy(ewD[slot], deg_s.at[colD[slot]],
                              ss[slot]).wait()

    didx_start(0, 0)

    @pl.loop(0, NCHD0 // 3)
    def _dblk(blk):
        for m in range(3):
            jj = blk * 3 + m
            nslot = (m + 1) % 3

            def _advance():
                _when(jj >= 2, lambda: dscat_wait(nslot))
                didx_start(nslot, jj + 1)

            _when(jj + 1 < NCHD0, _advance)
            didx_wait(m, jj)
            pltpu.async_copy(ewD[m], deg_s.at[colD[m]], ss[m], add=True)

    dscat_wait(0)
    dscat_wait(1)
    dscat_wait(2)
    plsc.subcore_barrier()
    pltpu.sync_copy(deg_s.at[pl.ds(s * ZRO, ZRO)],
                    degp_ref.at[pl.ds(c * NROWS_PAD + s * ZRO, ZRO)])


def _layer1_body(row_ref, col_ref, ew_ref, h_ref, degp_ref, agg_ref,
                 norm_ref,
                 acc_s, dinvv,
                 rowv0, rowv1, rowv2, rowv3,
                 colv0, colv1, colv2, colv3,
                 auxv0, auxv1, auxv2, auxv3,
                 normv0, normv1,
                 rows0, rows1, rows2, rows3,
                 zrows, zdeg, zdeg2,
                 si0, si1, si2, si3, sg0, sg1, sg2, sg3,
                 ss0, ss1, ss2, ss3, sn0, sn1):
    c = lax.axis_index("c")
    s = lax.axis_index("s")
    w = s * NC + c
    rowv = (rowv0, rowv1, rowv2, rowv3)
    colv = (colv0, colv1, colv2, colv3)
    auxv = (auxv0, auxv1, auxv2, auxv3)
    normv = (normv0, normv1)
    rows = (rows0, rows1, rows2, rows3)
    si = (si0, si1, si2, si3)
    sg = (sg0, sg1, sg2, sg3)
    ss = (ss0, ss1, ss2, ss3)
    sn = (sn0, sn1)

    for r in range(16):
        for k in range(NHID // 16):
            zrows[r, pl.ds(k * 16, 16)] = _Z16()

    for k in range(ZRO // 16):
        pltpu.async_copy(zrows, acc_s.at[pl.ds(s * ZRO + k * 16, 16)], sg0)

    @pl.loop(0, NS)
    def _piece(p):
        off = p * ZRO
        pltpu.sync_copy(degp_ref.at[pl.ds(off, ZRO)], zdeg)
        pltpu.sync_copy(degp_ref.at[pl.ds(NROWS_PAD + off, ZRO)], zdeg2)

        @pl.loop(0, ZRO // 16)
        def _newton(i):
            sl = pl.ds(i * 16, 16)
            dinvv[pl.ds(off + i * 16, 16)] = _rsqrt16(zdeg[sl] + zdeg2[sl])

    for k in range(ZRO // 16):
        pltpu.make_async_copy(zrows, acc_s.at[pl.ds(s * ZRO + k * 16, 16)],
                              sg0).wait()
    plsc.subcore_barrier()

    def compute(slot, jj, base):
        rv = rows[slot]
        nv = normv[slot % 2]
        for gi in range(C1 // 16):
            sl = pl.ds(gi * 16, 16)
            dr = plsc.load_gather(dinvv, [rowv[slot][sl]])
            dc = plsc.load_gather(dinvv, [colv[slot][sl]])
            nv[sl] = dr * auxv[slot][sl] * dc
        _scale_rows(rv, nv, C1, NHID)
        _when(jj >= 2,
              lambda: pltpu.make_async_copy(nv, norm_ref.at[pl.ds(base, C1)],
                                            sn[slot % 2]).wait())
        pltpu.async_copy(nv, norm_ref.at[pl.ds(base, C1)], sn[slot % 2])

    _make_agg_loop(row_ref, col_ref, ew_ref, h_ref, acc_s, rowv, colv,
                   auxv, rows, si, sg, ss, w, compute, C1)

    pltpu.make_async_copy(normv[0], norm_ref.at[pl.ds(0, C1)], sn[0]).wait()
    pltpu.make_async_copy(normv[1], norm_ref.at[pl.ds(0, C1)], sn[1]).wait()

    plsc.subcore_barrier()
    pltpu.sync_copy(acc_s.at[pl.ds(s * ZRO, ZRO)],
                    agg_ref.at[c, pl.ds(s * ZRO, ZRO)])


def _layer2_body(row_ref, col_ref, nm_ref, h_ref, agg_ref,
                 acc_s,
                 rowv0, rowv1, rowv2, rowv3,
                 colv0, colv1, colv2, colv3,
                 auxv0, auxv1, auxv2, auxv3,
                 rows0, rows1, rows2, rows3,
                 zrows,
                 si0, si1, si2, si3, sg0, sg1, sg2, sg3,
                 ss0, ss1, ss2, ss3):
    c = lax.axis_index("c")
    s = lax.axis_index("s")
    w = s * NC + c
    rowv = (rowv0, rowv1, rowv2, rowv3)
    colv = (colv0, colv1, colv2, colv3)
    auxv = (auxv0, auxv1, auxv2, auxv3)
    rows = (rows0, rows1, rows2, rows3)
    si = (si0, si1, si2, si3)
    sg = (sg0, sg1, sg2, sg3)
    ss = (ss0, ss1, ss2, ss3)

    for r in range(16):
        for k in range(NCLASS // 16):
            zrows[r, pl.ds(k * 16, 16)] = _Z16()
    for k in range(ZRO // 16):
        pltpu.async_copy(zrows, acc_s.at[pl.ds(s * ZRO + k * 16, 16)], sg0)
    for k in range(ZRO // 16):
        pltpu.make_async_copy(zrows, acc_s.at[pl.ds(s * ZRO + k * 16, 16)],
                              sg0).wait()
    plsc.subcore_barrier()

    def compute(slot, jj, base):
        _scale_rows(rows[slot], auxv[slot], C2, NCLASS)

    _make_agg_loop(row_ref, col_ref, nm_ref, h_ref, acc_s, rowv, colv,
                   auxv, rows, si, sg, ss, w, compute, C2)

    plsc.subcore_barrier()
    pltpu.sync_copy(acc_s.at[pl.ds(s * ZRO, ZRO)],
                    agg_ref.at[c, pl.ds(s * ZRO, ZRO)])


_sc_mesh = plsc.VectorSubcoreMesh(core_axis_name="c", subcore_axis_name="s")

_DMA = pltpu.SemaphoreType.DMA

_sc_deg = functools.partial(
    pl.kernel,
    out_type=jax.ShapeDtypeStruct((NC * NROWS_PAD,), jnp.float32),
    mesh=_sc_mesh,
    compiler_params=pltpu.CompilerParams(needs_layout_passes=False),
    scratch_types=[
        pltpu.VMEM_SHARED((NROWS_PAD,), jnp.float32),
    ] + [pltpu.VMEM((CD,), jnp.int32)] * 3
      + [pltpu.VMEM((CD,), jnp.float32)] * 3
      + [pltpu.VMEM((ZRO,), jnp.float32)]
      + [_DMA] * 6,
)(_deg_body)

_sc_layer1 = functools.partial(
    pl.kernel,
    out_type=(jax.ShapeDtypeStruct((NC, NROWS_PAD, NHID), jnp.float32),
              jax.ShapeDtypeStruct((E_PAD,), jnp.float32)),
    mesh=_sc_mesh,
    compiler_params=pltpu.CompilerParams(needs_layout_passes=False),
    scratch_types=[
        pltpu.VMEM_SHARED((NROWS_PAD, NHID), jnp.float32),
        pltpu.VMEM((NROWS_PAD,), jnp.float32),
    ] + [pltpu.VMEM((C1,), jnp.int32)] * 4
      + [pltpu.VMEM((C1,), jnp.int32)] * 4
      + [pltpu.VMEM((C1,), jnp.float32)] * 4
      + [pltpu.VMEM((C1,), jnp.float32)] * 2
      + [pltpu.VMEM((C1, NHID), jnp.float32)] * 4
      + [pltpu.VMEM((16, NHID), jnp.float32),
         pltpu.VMEM((ZRO,), jnp.float32),
         pltpu.VMEM((ZRO,), jnp.float32)]
      + [_DMA] * 14,
)(_layer1_body)

_sc_layer2 = functools.partial(
    pl.kernel,
    out_type=jax.ShapeDtypeStruct((NC, NROWS_PAD, NCLASS), jnp.float32),
    mesh=_sc_mesh,
    compiler_params=pltpu.CompilerParams(needs_layout_passes=False,
                                         use_tc_tiling_on_sc=False),
    scratch_types=[
        pltpu.VMEM_SHARED((NROWS_PAD, NCLASS), jnp.float32),
    ] + [pltpu.VMEM((C2,), jnp.int32)] * 4
      + [pltpu.VMEM((C2,), jnp.int32)] * 4
      + [pltpu.VMEM((C2,), jnp.float32)] * 4
      + [pltpu.VMEM((C2, NCLASS), jnp.float32)] * 4
      + [pltpu.VMEM((16, NCLASS), jnp.float32)]
      + [_DMA] * 12,
)(_layer2_body)


def _mm_body(x_ref, w_ref, o_ref):
    o_ref[...] = jnp.dot(x_ref[...], w_ref[...],
                         preferred_element_type=jnp.float32)


def _matmul(x, w):
    m, k = x.shape
    n = w.shape[1]
    return pl.pallas_call(
        _mm_body,
        grid=(m // MROWS,),
        in_specs=[pl.BlockSpec((MROWS, k), lambda i: (i, 0)),
                  pl.BlockSpec((k, n), lambda i: (0, 0))],
        out_specs=pl.BlockSpec((MROWS, n), lambda i: (i, 0)),
        out_shape=jax.ShapeDtypeStruct((m, n), jnp.float32),
    )(x, w)


def _relu_mm_body(p0_ref, p1_ref, b_ref, w_ref, o_ref):
    h = jnp.maximum(p0_ref[...] + p1_ref[...] + b_ref[...], 0.0)
    o_ref[...] = jnp.dot(h, w_ref[...], preferred_element_type=jnp.float32)


def _relu_mm(p0, p1, b, w):
    m, k = p0.shape
    n = w.shape[1]
    return pl.pallas_call(
        _relu_mm_body,
        grid=(m // MROWS,),
        in_specs=[pl.BlockSpec((MROWS, k), lambda i: (i, 0)),
                  pl.BlockSpec((MROWS, k), lambda i: (i, 0)),
                  pl.BlockSpec((1, k), lambda i: (0, 0)),
                  pl.BlockSpec((k, n), lambda i: (0, 0))],
        out_specs=pl.BlockSpec((MROWS, n), lambda i: (i, 0)),
        out_shape=jax.ShapeDtypeStruct((m, n), jnp.float32),
    )(p0, p1, b, w)


def _lsm_body(p0_ref, p1_ref, b_ref, o_ref):
    sv = p0_ref[...] + p1_ref[...] + b_ref[...]
    m = jnp.max(sv, axis=1, keepdims=True)
    t = sv - m
    o_ref[...] = t - jnp.log(jnp.sum(jnp.exp(t), axis=1, keepdims=True))


def _logsoftmax(p0, p1, b):
    m, n = p0.shape
    return pl.pallas_call(
        _lsm_body,
        grid=(m // MROWS,),
        in_specs=[pl.BlockSpec((MROWS, n), lambda i: (i, 0)),
                  pl.BlockSpec((MROWS, n), lambda i: (i, 0)),
                  pl.BlockSpec((1, n), lambda i: (0, 0))],
        out_specs=pl.BlockSpec((MROWS, n), lambda i: (i, 0)),
        out_shape=jax.ShapeDtypeStruct((m, n), jnp.float32),
    )(p0, p1, b)


def kernel(x, edge_index, edge_weight, W1, b1, W2, b2):
    row = edge_index[0].astype(jnp.int32)
    col = edge_index[1].astype(jnp.int32)
    loop_idx = jnp.arange(N, dtype=jnp.int32)
    pad = E_PAD - E - N
    zpad = jnp.zeros((pad,), jnp.int32)
    row_ext = jnp.concatenate([row, loop_idx, zpad])
    col_ext = jnp.concatenate([col, loop_idx, zpad])
    ew_ext = jnp.concatenate([edge_weight.astype(jnp.float32),
                              jnp.ones((N,), jnp.float32),
                              jnp.zeros((pad,), jnp.float32)])

    degp = _sc_deg(col_ext, ew_ext)
    h1 = _matmul(x, W1)
    agg1, norm = _sc_layer1(row_ext, col_ext, ew_ext, h1, degp)
    h2 = _relu_mm(agg1[0, :N], agg1[1, :N], b1.reshape(1, NHID), W2)
    agg2 = _sc_layer2(row_ext, col_ext, norm, h2)
    return _logsoftmax(agg2[0, :N], agg2[1, :N], b2.reshape(1, NCLASS))
